# baseline (speedup 1.0000x reference)
import jax
import jax.numpy as jnp
from jax.experimental import pallas as pl
from jax.experimental.pallas import tpu as pltpu

_B, _T, _R, _C = 4, 8192, 129, 2
_TT, _TI = _T // 128, 128
_ROWS = _B * _R * _TT * _C
_BLOCK_ROWS = 1536


def _row_tables(src_token):
    t = jnp.asarray(src_token, jnp.int32)
    row = jnp.arange(_ROWS, dtype=jnp.int32)
    reg = (row // (_TT * _C)) % _R
    col0 = (row % _C) == 0

    is_start = t == 0
    is_digit = (t >= 1) & (t <= 10)
    is_plus = t == 11
    is_minus = t == 12
    is_equals = t == 13
    digit_val = (t - 1) % 10

    digit_band = (reg >= 2) & (reg <= 11) & col0
    digit_hit = (reg == 2 + (digit_val % 10)) & col0
    op_reg = (reg == 1) & col0
    result_regs = (reg >= 14) & (reg <= 16) & col0

    m = jnp.zeros((_ROWS,), jnp.bool_)
    v = jnp.zeros((_ROWS,), jnp.float32)
    m = m | (is_start & (reg < 20))
    m = m | (is_digit & digit_band)
    v = jnp.where(is_digit & digit_hit, 1.0, v)
    m = m | (is_plus & op_reg)
    v = jnp.where(is_plus & op_reg, 1.0, v)
    m = m | (is_minus & op_reg)
    v = jnp.where(is_minus & op_reg, -1.0, v)
    m = m | (is_equals & (result_regs | op_reg | digit_band))
    return m.astype(jnp.float32)[:, None], v[:, None]


def _body(x_ref, m_ref, v_ref, o_ref):
    o_ref[...] = jnp.where(m_ref[...] != 0.0, v_ref[...], x_ref[...])


def kernel(carrier_freq, src_token, tgt_token):
    x2d = (
        carrier_freq.transpose(0, 2, 1, 3)
        .reshape(_B, _R, _TT, _TI, _C)
        .transpose(0, 1, 2, 4, 3)
        .reshape(_ROWS, _TI)
    )
    mask, val = _row_tables(src_token)
    out = pl.pallas_call(
        _body,
        grid=(_ROWS // _BLOCK_ROWS,),
        in_specs=[
            pl.BlockSpec((_BLOCK_ROWS, _TI), lambda i: (i, 0)),
            pl.BlockSpec((_BLOCK_ROWS, 1), lambda i: (i, 0)),
            pl.BlockSpec((_BLOCK_ROWS, 1), lambda i: (i, 0)),
        ],
        out_specs=pl.BlockSpec((_BLOCK_ROWS, _TI), lambda i: (i, 0)),
        out_shape=jax.ShapeDtypeStruct((_ROWS, _TI), jnp.float32),
        compiler_params=pltpu.CompilerParams(
            dimension_semantics=("parallel",),
        ),
    )(x2d, mask, val)
    return (
        out.reshape(_B, _R, _TT, _C, _TI)
        .transpose(0, 1, 2, 4, 3)
        .reshape(_B, _R, _T, _C)
        .transpose(0, 2, 1, 3)
    )


# speedup vs baseline: 1.0885x; 1.0885x over previous
import jax
import jax.numpy as jnp
from jax.experimental import pallas as pl
from jax.experimental.pallas import tpu as pltpu

_B, _T, _R, _C = 4, 8192, 129, 2
_TT, _TI = _T // 128, 128
_ROWS = _B * _R * _TT * _C
_BLOCK_ROWS = 8256


def _row_tables(src_token):
    t = jnp.asarray(src_token, jnp.int32)
    row = jnp.arange(_ROWS, dtype=jnp.int32)
    reg = (row // (_TT * _C)) % _R
    col0 = (row % _C) == 0

    is_start = t == 0
    is_digit = (t >= 1) & (t <= 10)
    is_plus = t == 11
    is_minus = t == 12
    is_equals = t == 13
    digit_val = (t - 1) % 10

    digit_band = (reg >= 2) & (reg <= 11) & col0
    digit_hit = (reg == 2 + (digit_val % 10)) & col0
    op_reg = (reg == 1) & col0
    result_regs = (reg >= 14) & (reg <= 16) & col0

    m = jnp.zeros((_ROWS,), jnp.bool_)
    v = jnp.zeros((_ROWS,), jnp.float32)
    m = m | (is_start & (reg < 20))
    m = m | (is_digit & digit_band)
    v = jnp.where(is_digit & digit_hit, 1.0, v)
    m = m | (is_plus & op_reg)
    v = jnp.where(is_plus & op_reg, 1.0, v)
    m = m | (is_minus & op_reg)
    v = jnp.where(is_minus & op_reg, -1.0, v)
    m = m | (is_equals & (result_regs | op_reg | digit_band))
    return m.astype(jnp.float32)[:, None], v[:, None]


def _body(x_ref, m_ref, v_ref, o_ref):
    o_ref[...] = jnp.where(m_ref[...] != 0.0, v_ref[...], x_ref[...])


def kernel(carrier_freq, src_token, tgt_token):
    x2d = (
        carrier_freq.transpose(0, 2, 1, 3)
        .reshape(_B, _R, _TT, _TI, _C)
        .transpose(0, 1, 2, 4, 3)
        .reshape(_ROWS, _TI)
    )
    mask, val = _row_tables(src_token)
    out = pl.pallas_call(
        _body,
        grid=(_ROWS // _BLOCK_ROWS,),
        in_specs=[
            pl.BlockSpec((_BLOCK_ROWS, _TI), lambda i: (i, 0)),
            pl.BlockSpec((_BLOCK_ROWS, 1), lambda i: (i, 0)),
            pl.BlockSpec((_BLOCK_ROWS, 1), lambda i: (i, 0)),
        ],
        out_specs=pl.BlockSpec((_BLOCK_ROWS, _TI), lambda i: (i, 0)),
        out_shape=jax.ShapeDtypeStruct((_ROWS, _TI), jnp.float32),
        compiler_params=pltpu.CompilerParams(
            dimension_semantics=("parallel",),
        ),
    )(x2d, mask, val)
    return (
        out.reshape(_B, _R, _TT, _C, _TI)
        .transpose(0, 1, 2, 4, 3)
        .reshape(_B, _R, _T, _C)
        .transpose(0, 2, 1, 3)
    )
